# baseline (speedup 1.0000x reference)
import functools

import jax
import jax.numpy as jnp
from jax import lax
from jax.experimental import pallas as pl
from jax.experimental.pallas import tpu as pltpu
from jax.experimental.pallas import tpu_sc as plsc

SEQ = 200
WORD_DIM = 64
HIDDEN = 128
L = 16
ROWS = 8


def _body(sent_hbm, word_hbm, pos_hbm, wtr_hbm, b_hbm, out_hbm, hid_hbm,
          idx_v, rows_v, pos_v, outblk_v, psum_v, psums_v, wt_v, bvec_v,
          hidout_v, shared, sem):
    c = lax.axis_index("c")
    s = lax.axis_index("s")

    @pl.when(c == 0)
    def _gather_phase():
        accw = [jnp.zeros((L,), jnp.float32) for _ in range(4)]
        accp = [jnp.zeros((L,), jnp.float32) for _ in range(4)]
        for rnd in range(2):
            base = rnd * 128 + s * ROWS
            active = (base + ROWS) <= SEQ

            @pl.when(active)
            def _dma_in():
                pltpu.sync_copy(sent_hbm.at[pl.ds(base, ROWS)], idx_v)
                pltpu.async_copy(word_hbm.at[idx_v], rows_v, sem).wait()
                pltpu.sync_copy(pos_hbm.at[pl.ds(base, ROWS)], pos_v)

            for r in range(ROWS):
                for ch in range(4):
                    wv = rows_v[r, pl.ds(ch * L, L)]
                    pv = pos_v[r, pl.ds(ch * L, L)]
                    if rnd == 1:
                        wv = jnp.where(active, wv, 0.0)
                        pv = jnp.where(active, pv, 0.0)
                    outblk_v[r, pl.ds(ch * L, L)] = wv
                    outblk_v[r, pl.ds(WORD_DIM + ch * L, L)] = pv
                    accw[ch] = accw[ch] + wv
                    accp[ch] = accp[ch] + pv

            @pl.when(active)
            def _dma_out():
                pltpu.sync_copy(outblk_v, out_hbm.at[pl.ds(base, ROWS)])

        for ch in range(4):
            psum_v[0, pl.ds(ch * L, L)] = accw[ch]
            psum_v[0, pl.ds(WORD_DIM + ch * L, L)] = accp[ch]
        pltpu.sync_copy(psum_v, shared.at[pl.ds(s, 1)])

    plsc.subcore_barrier()

    @pl.when((c == 0) & (s < 8))
    def _linear_phase():
        pltpu.sync_copy(shared, psums_v)
        pltpu.sync_copy(wtr_hbm.at[pl.ds(s, 1)], wt_v)
        pltpu.sync_copy(b_hbm.at[pl.ds(s * L, L)], bvec_v)
        totals = []
        for kc in range(8):
            t = jnp.zeros((L,), jnp.float32)
            for w in range(16):
                t = t + psums_v[w, pl.ds(kc * L, L)]
            totals.append(t * (1.0 / SEQ))
        dnums = lax.GatherDimensionNumbers(
            offset_dims=(), collapsed_slice_dims=(0,), start_index_map=(0,))
        acc = bvec_v[...]
        for k in range(HIDDEN):
            lane = jnp.full((L, 1), k % L, jnp.int32)
            scal = lax.gather(totals[k // L], lane, dnums, (1,),
                              mode=lax.GatherScatterMode.PROMISE_IN_BOUNDS)
            acc = acc + scal * wt_v[0, pl.ds(k * L, L)]
        hidout_v[...] = acc
        pltpu.sync_copy(hidout_v, hid_hbm.at[pl.ds(s * L, L)])


@functools.partial(jax.jit, static_argnames=())
def _encode(sent, word_table, pos_table, wtr, b):
    mesh = plsc.VectorSubcoreMesh(core_axis_name="c", subcore_axis_name="s")
    run = functools.partial(
        pl.kernel,
        mesh=mesh,
        compiler_params=pltpu.CompilerParams(use_tc_tiling_on_sc=False),
        out_type=[
            jax.ShapeDtypeStruct((SEQ, HIDDEN), jnp.float32),
            jax.ShapeDtypeStruct((HIDDEN,), jnp.float32),
        ],
        scratch_types=[
            pltpu.VMEM((ROWS,), jnp.int32),
            pltpu.VMEM((ROWS, WORD_DIM), jnp.float32),
            pltpu.VMEM((ROWS, WORD_DIM), jnp.float32),
            pltpu.VMEM((ROWS, HIDDEN), jnp.float32),
            pltpu.VMEM((1, HIDDEN), jnp.float32),
            pltpu.VMEM((16, HIDDEN), jnp.float32),
            pltpu.VMEM((1, 8 * HIDDEN * L // 8), jnp.float32),
            pltpu.VMEM((L,), jnp.float32),
            pltpu.VMEM((L,), jnp.float32),
            pltpu.VMEM_SHARED((16, HIDDEN), jnp.float32),
            pltpu.SemaphoreType.DMA,
        ],
    )(_body)
    return run(sent, word_table, pos_table, wtr, b)


def kernel(sentence, word_table, pos_table, W, b):
    sent = sentence.astype(jnp.int32)
    wtr = W.reshape(8, L, HIDDEN).transpose(0, 2, 1).reshape(8, HIDDEN * L)
    out, hid = _encode(sent, word_table, pos_table, wtr, b)
    return out.reshape(SEQ, 1, HIDDEN), hid.reshape(1, 1, HIDDEN)
